# baseline (speedup 1.0000x reference)
import dataclasses
import functools

import jax
import jax.numpy as jnp
from jax import lax
from jax.experimental import pallas as pl
from jax.experimental.pallas import tpu as pltpu
from jax.experimental.pallas import tpu_sc as plsc

B, S, D = 8, 1024, 256
L, K = 4, 4096
N = B * S
COMMIT = 0.25
RBLK = 512
NRB = N // RBLK
SC_CORES = 2
SC_SUBCORES = 16
SC_WORKERS = SC_CORES * SC_SUBCORES
ROWS_PER_W = N // SC_WORKERS
MM_PREC = None


def _cn_into_scratch(cb, cn_scr):
    sq = cb * cb
    ones = jnp.ones((8, D), jnp.float32)
    r = lax.dot_general(ones, sq, (((1,), (1,)), ((), ())),
                        preferred_element_type=jnp.float32,
                        precision=lax.Precision.HIGHEST)
    cn_scr[...] = r[0:1, :]


def _argmin_tail(res, cb, cn_scr, idx_ref, lsum_ref, step):
    @pl.when(step == 0)
    def _():
        _cn_into_scratch(cb, cn_scr)

    sm2 = lax.dot_general(res * -2.0, cb, (((1,), (1,)), ((), ())),
                          preferred_element_type=jnp.float32,
                          precision=MM_PREC)
    rn = jnp.sum(res * res, axis=1, keepdims=True)
    d = (rn + sm2) + cn_scr[...]
    mv = jnp.min(d, axis=1, keepdims=True)
    ii = lax.broadcasted_iota(jnp.int32, (1, K), 1).astype(jnp.float32)
    idxf = jnp.min(jnp.where(d == mv, ii, float(K)), axis=1, keepdims=True)
    idx_ref[...] = idxf.astype(jnp.int32).reshape(1, 1, RBLK)
    part = jnp.sum(mv, keepdims=True)

    @pl.when(step == 0)
    def _():
        lsum_ref[...] = part

    @pl.when(step != 0)
    def _():
        lsum_ref[...] = lsum_ref[...] + part


def _layer0_body(z_ref, cb_ref, idx_ref, lsum_ref, cn_scr):
    _argmin_tail(z_ref[...], cb_ref[0], cn_scr, idx_ref, lsum_ref,
                 pl.program_id(0))


def _layer1_body(res_ref, zq_ref, cb_ref,
                 resout_ref, qacc_ref, idx_ref, lsum_ref, cn_scr):
    rp = res_ref[...]
    q = rp + (zq_ref[...] - rp)
    qacc_ref[...] = q
    res = rp - q
    resout_ref[...] = res
    _argmin_tail(res, cb_ref[0], cn_scr, idx_ref, lsum_ref,
                 pl.program_id(0))


def _layerN_body(res_ref, zq_ref, qin_ref, cb_ref,
                 resout_ref, qacc_ref, idx_ref, lsum_ref, cn_scr):
    rp = res_ref[...]
    q = rp + (zq_ref[...] - rp)
    qacc_ref[...] = qin_ref[...] + q
    res = rp - q
    resout_ref[...] = res
    _argmin_tail(res, cb_ref[0], cn_scr, idx_ref, lsum_ref,
                 pl.program_id(0))


_ROWSPEC = pl.BlockSpec((RBLK, D), lambda i: (i, 0))
_IDXSPEC = pl.BlockSpec((1, 1, RBLK), lambda i: (i, 0, 0))
_SCALSPEC = pl.BlockSpec((1, 1), lambda i: (0, 0))

_IDX_SHAPE = jax.ShapeDtypeStruct((NRB, 1, RBLK), jnp.int32)
_SCAL_SHAPE = jax.ShapeDtypeStruct((1, 1), jnp.float32)
_ROW_SHAPE = jax.ShapeDtypeStruct((N, D), jnp.float32)
_CN_SCRATCH = [pltpu.VMEM((1, K), jnp.float32)]


def _cbspec(lidx):
    return pl.BlockSpec((1, K, D), lambda i, _l=lidx: (_l, 0, 0))


def _layer0(z, codebooks):
    return pl.pallas_call(
        _layer0_body,
        grid=(NRB,),
        in_specs=[_ROWSPEC, _cbspec(0)],
        out_specs=(_IDXSPEC, _SCALSPEC),
        out_shape=(_IDX_SHAPE, _SCAL_SHAPE),
        scratch_shapes=_CN_SCRATCH,
    )(z, codebooks)


def _layer1(res, zq, codebooks):
    return pl.pallas_call(
        _layer1_body,
        grid=(NRB,),
        in_specs=[_ROWSPEC, _ROWSPEC, _cbspec(1)],
        out_specs=(_ROWSPEC, _ROWSPEC, _IDXSPEC, _SCALSPEC),
        out_shape=(_ROW_SHAPE, _ROW_SHAPE, _IDX_SHAPE, _SCAL_SHAPE),
        scratch_shapes=_CN_SCRATCH,
    )(res, zq, codebooks)


def _layerN(res, zq, qacc, codebooks, lidx):
    return pl.pallas_call(
        _layerN_body,
        grid=(NRB,),
        in_specs=[_ROWSPEC, _ROWSPEC, _ROWSPEC, _cbspec(lidx)],
        out_specs=(_ROWSPEC, _ROWSPEC, _IDXSPEC, _SCALSPEC),
        out_shape=(_ROW_SHAPE, _ROW_SHAPE, _IDX_SHAPE, _SCAL_SHAPE),
        scratch_shapes=_CN_SCRATCH,
    )(res, zq, qacc, codebooks)


def _sc_gather_hist(cb, idx2d):
    mesh = plsc.VectorSubcoreMesh(core_axis_name="c", subcore_axis_name="s")
    cp = pltpu.CompilerParams()
    if "needs_layout_passes" in pltpu.CompilerParams.__dataclass_fields__:
        cp = dataclasses.replace(cp, needs_layout_passes=False)

    @functools.partial(
        pl.kernel,
        compiler_params=cp,
        out_type=(jax.ShapeDtypeStruct((N, D), jnp.float32),
                  jax.ShapeDtypeStruct((SC_CORES, K // 16, 16), jnp.float32)),
        mesh=mesh,
        scratch_types=[
            pltpu.VMEM((2, 128), jnp.int32),
            pltpu.VMEM((ROWS_PER_W, D), jnp.float32),
            pltpu.VMEM((K // 16, 16), jnp.float32),
            pltpu.VMEM((2, 128), jnp.int32),
            pltpu.VMEM_SHARED((K // 16, 16), jnp.float32),
            pltpu.SemaphoreType.DMA,
        ],
    )
    def k(cb_hbm, idx_hbm, zq_hbm, cnt_hbm, idx_v, rows_v, cnt_v, ridx_v,
          cnt_sh, sem):
        cid = lax.axis_index("c")
        sid = lax.axis_index("s")
        wid = sid * SC_CORES + cid
        base = wid * ROWS_PER_W

        pltpu.sync_copy(idx_hbm.at[pl.ds(wid * 2, 2)], idx_v)
        for j in range(2):
            pltpu.async_copy(cb_hbm.at[idx_v.at[j]],
                             rows_v.at[pl.ds(j * 128, 128)], sem).wait()
        pltpu.sync_copy(rows_v, zq_hbm.at[pl.ds(base, ROWS_PER_W)])

        @pl.loop(0, K // 16)
        def _(r):
            cnt_v[r] = jnp.zeros((16,), jnp.float32)

        @pl.when(sid == 0)
        def _():
            pltpu.sync_copy(cnt_v, cnt_sh)
        plsc.subcore_barrier()

        ones = jnp.ones((16,), jnp.float32)
        for j in range(2):
            for t in range(8):
                v = idx_v[j, pl.ds(t * 16, 16)]
                row = lax.shift_right_logical(v, 4)
                lane = lax.bitwise_and(v, 15)
                plsc.addupdate_scatter(cnt_v, [row, lane], ones)
            for t in range(8):
                ridx_v[j, pl.ds(t * 16, 16)] = (
                    lax.iota(jnp.int32, 16) + (j * 128 + t * 16))

        for j in range(2):
            pltpu.sync_copy(cnt_v.at[pl.ds(j * 128, 128)],
                            cnt_sh.at[ridx_v.at[j]], add=True)
        plsc.subcore_barrier()

        @pl.when(sid == 0)
        def _():
            pltpu.sync_copy(cnt_sh, cnt_hbm.at[cid])

    return k(cb, idx2d)


def _final_body(res_ref, zq_ref, qin_ref, cnt_ref, ls_ref,
                qsum_ref, commit_ref, perp_ref):
    i = pl.program_id(0)
    rp = res_ref[...]
    q = rp + (zq_ref[...] - rp)
    qsum_ref[...] = qin_ref[...] + q

    @pl.when(i == NRB - 1)
    def _():
        cnt = cnt_ref[...]
        counts = cnt[:, 0, :] + cnt[:, 1, :]
        probs = counts * (1.0 / N)
        ent = jnp.sum(probs * jnp.log(probs + 1e-10), axis=1, keepdims=True)
        perps = jnp.exp(-ent)
        perp_ref[...] = jnp.sum(perps, keepdims=True) * (1.0 / L)
        ls = ls_ref[...] * (COMMIT / (N * D))
        commit_ref[...] = jnp.sum(ls, keepdims=True)

    @pl.when(i == 0)
    def _():
        commit_ref[...] = jnp.zeros((1, 1), jnp.float32)
        perp_ref[...] = jnp.zeros((1, 1), jnp.float32)


def _final(res, zq, qacc, counts, lsums):
    return pl.pallas_call(
        _final_body,
        grid=(NRB,),
        in_specs=[
            _ROWSPEC, _ROWSPEC, _ROWSPEC,
            pl.BlockSpec((L, SC_CORES, K), lambda i: (0, 0, 0)),
            pl.BlockSpec((1, L), lambda i: (0, 0)),
        ],
        out_specs=(_ROWSPEC, _SCALSPEC, _SCALSPEC),
        out_shape=(_ROW_SHAPE, _SCAL_SHAPE, _SCAL_SHAPE),
    )(res, zq, qacc, counts, lsums)


def kernel(z, codebooks):
    zf = z.reshape(N, D)

    idx0, ls0 = _layer0(zf, codebooks)
    zq0, cnt0 = _sc_gather_hist(codebooks[0], idx0.reshape(N // 128, 128))

    res1, qacc1, idx1, ls1 = _layer1(zf, zq0, codebooks)
    zq1, cnt1 = _sc_gather_hist(codebooks[1], idx1.reshape(N // 128, 128))

    res2, qacc2, idx2, ls2 = _layerN(res1, zq1, qacc1, codebooks, 2)
    zq2, cnt2 = _sc_gather_hist(codebooks[2], idx2.reshape(N // 128, 128))

    res3, qacc3, idx3, ls3 = _layerN(res2, zq2, qacc2, codebooks, 3)
    zq3, cnt3 = _sc_gather_hist(codebooks[3], idx3.reshape(N // 128, 128))

    counts = jnp.stack([cnt0, cnt1, cnt2, cnt3]).reshape(L, SC_CORES, K)
    lsums = jnp.concatenate([ls0, ls1, ls2, ls3], axis=1)

    qsum, commit, perp = _final(res3, zq3, qacc3, counts, lsums)

    quantized_sum = qsum.reshape(B, S, D)
    indices = jnp.stack(
        [idx0.reshape(B, S), idx1.reshape(B, S),
         idx2.reshape(B, S), idx3.reshape(B, S)], axis=-1)
    total_commitment = commit[0, 0]
    codebook_loss = jnp.zeros((), jnp.float32)
    avg_perplexity = perp[0, 0]
    return (quantized_sum, indices, total_commitment, codebook_loss,
            avg_perplexity)


# speedup vs baseline: 1.0597x; 1.0597x over previous
import dataclasses
import functools

import jax
import jax.numpy as jnp
from jax import lax
from jax.experimental import pallas as pl
from jax.experimental.pallas import tpu as pltpu
from jax.experimental.pallas import tpu_sc as plsc

B, S, D = 8, 1024, 256
L, K = 4, 4096
N = B * S
COMMIT = 0.25
RBLK = 512
NRB = N // RBLK
SC_CORES = 2
SC_SUBCORES = 16
SC_WORKERS = SC_CORES * SC_SUBCORES
ROWS_PER_W = N // SC_WORKERS
MM_PREC = None


def _cn_into_scratch(cb, cn_scr):
    sq = cb * cb
    ones = jnp.ones((8, D), jnp.float32)
    r = lax.dot_general(ones, sq, (((1,), (1,)), ((), ())),
                        preferred_element_type=jnp.float32,
                        precision=lax.Precision.HIGHEST)
    cn_scr[...] = r[0:1, :]


def _argmin_tail(res, cb, cn_scr, idx_ref, lsum_ref, step):
    @pl.when(step == 0)
    def _():
        _cn_into_scratch(cb, cn_scr)

    sm2 = lax.dot_general(res * -2.0, cb, (((1,), (1,)), ((), ())),
                          preferred_element_type=jnp.float32,
                          precision=MM_PREC)
    rn = jnp.sum(res * res, axis=1, keepdims=True)
    cn = cn_scr[...]
    ii = lax.broadcasted_iota(jnp.int32, (1, K), 1).astype(jnp.float32)
    CH = 512
    mv = None
    idxf = None
    for c in range(0, K, CH):
        dc = (rn + sm2[:, c:c + CH]) + cn[:, c:c + CH]
        mc = jnp.min(dc, axis=1, keepdims=True)
        ic = jnp.min(jnp.where(dc == mc, ii[:, c:c + CH], float(K)),
                     axis=1, keepdims=True)
        if mv is None:
            mv, idxf = mc, ic
        else:
            upd = mc < mv
            mv = jnp.where(upd, mc, mv)
            idxf = jnp.where(upd, ic, idxf)
    idx_ref[...] = idxf.astype(jnp.int32).reshape(1, 1, RBLK)
    part = jnp.sum(mv, keepdims=True)

    @pl.when(step == 0)
    def _():
        lsum_ref[...] = part

    @pl.when(step != 0)
    def _():
        lsum_ref[...] = lsum_ref[...] + part


def _layer0_body(z_ref, cb_ref, idx_ref, lsum_ref, cn_scr):
    _argmin_tail(z_ref[...], cb_ref[0], cn_scr, idx_ref, lsum_ref,
                 pl.program_id(0))


def _layer1_body(res_ref, zq_ref, cb_ref,
                 resout_ref, qacc_ref, idx_ref, lsum_ref, cn_scr):
    rp = res_ref[...]
    q = rp + (zq_ref[...] - rp)
    qacc_ref[...] = q
    res = rp - q
    resout_ref[...] = res
    _argmin_tail(res, cb_ref[0], cn_scr, idx_ref, lsum_ref,
                 pl.program_id(0))


def _layerN_body(res_ref, zq_ref, qin_ref, cb_ref,
                 resout_ref, qacc_ref, idx_ref, lsum_ref, cn_scr):
    rp = res_ref[...]
    q = rp + (zq_ref[...] - rp)
    qacc_ref[...] = qin_ref[...] + q
    res = rp - q
    resout_ref[...] = res
    _argmin_tail(res, cb_ref[0], cn_scr, idx_ref, lsum_ref,
                 pl.program_id(0))


_ROWSPEC = pl.BlockSpec((RBLK, D), lambda i: (i, 0))
_IDXSPEC = pl.BlockSpec((1, 1, RBLK), lambda i: (i, 0, 0))
_SCALSPEC = pl.BlockSpec((1, 1), lambda i: (0, 0))

_IDX_SHAPE = jax.ShapeDtypeStruct((NRB, 1, RBLK), jnp.int32)
_SCAL_SHAPE = jax.ShapeDtypeStruct((1, 1), jnp.float32)
_ROW_SHAPE = jax.ShapeDtypeStruct((N, D), jnp.float32)
_CN_SCRATCH = [pltpu.VMEM((1, K), jnp.float32)]


def _cbspec(lidx):
    return pl.BlockSpec((1, K, D), lambda i, _l=lidx: (_l, 0, 0))


def _layer0(z, codebooks):
    return pl.pallas_call(
        _layer0_body,
        grid=(NRB,),
        in_specs=[_ROWSPEC, _cbspec(0)],
        out_specs=(_IDXSPEC, _SCALSPEC),
        out_shape=(_IDX_SHAPE, _SCAL_SHAPE),
        scratch_shapes=_CN_SCRATCH,
    )(z, codebooks)


def _layer1(res, zq, codebooks):
    return pl.pallas_call(
        _layer1_body,
        grid=(NRB,),
        in_specs=[_ROWSPEC, _ROWSPEC, _cbspec(1)],
        out_specs=(_ROWSPEC, _ROWSPEC, _IDXSPEC, _SCALSPEC),
        out_shape=(_ROW_SHAPE, _ROW_SHAPE, _IDX_SHAPE, _SCAL_SHAPE),
        scratch_shapes=_CN_SCRATCH,
    )(res, zq, codebooks)


def _layerN(res, zq, qacc, codebooks, lidx):
    return pl.pallas_call(
        _layerN_body,
        grid=(NRB,),
        in_specs=[_ROWSPEC, _ROWSPEC, _ROWSPEC, _cbspec(lidx)],
        out_specs=(_ROWSPEC, _ROWSPEC, _IDXSPEC, _SCALSPEC),
        out_shape=(_ROW_SHAPE, _ROW_SHAPE, _IDX_SHAPE, _SCAL_SHAPE),
        scratch_shapes=_CN_SCRATCH,
    )(res, zq, qacc, codebooks)


def _sc_gather_hist(cb_flat, idx2d, lidx):
    mesh = plsc.VectorSubcoreMesh(core_axis_name="c", subcore_axis_name="s")
    cp = pltpu.CompilerParams()
    if "needs_layout_passes" in pltpu.CompilerParams.__dataclass_fields__:
        cp = dataclasses.replace(cp, needs_layout_passes=False)

    @functools.partial(
        pl.kernel,
        compiler_params=cp,
        out_type=(jax.ShapeDtypeStruct((N, D), jnp.float32),
                  jax.ShapeDtypeStruct((SC_CORES, K // 16, 16), jnp.float32)),
        mesh=mesh,
        scratch_types=[
            pltpu.VMEM((2, 128), jnp.int32),
            pltpu.VMEM((ROWS_PER_W, D), jnp.float32),
            pltpu.VMEM((K // 16, 16), jnp.float32),
            pltpu.VMEM((2, 128), jnp.int32),
            pltpu.VMEM_SHARED((K // 16, 16), jnp.float32),
            pltpu.SemaphoreType.DMA,
        ],
    )
    def k(cb_hbm, idx_hbm, zq_hbm, cnt_hbm, idx_v, rows_v, cnt_v, ridx_v,
          cnt_sh, sem):
        cid = lax.axis_index("c")
        sid = lax.axis_index("s")
        wid = sid * SC_CORES + cid
        base = wid * ROWS_PER_W

        pltpu.sync_copy(idx_hbm.at[pl.ds(wid * 2, 2)], idx_v)
        for j in range(2):
            for t in range(8):
                sl = pl.ds(t * 16, 16)
                idx_v[j, sl] = idx_v[j, sl] + (lidx * K)
        copies = [
            pltpu.async_copy(cb_hbm.at[idx_v.at[j]],
                             rows_v.at[pl.ds(j * 128, 128)], sem)
            for j in range(2)
        ]

        @pl.loop(0, K // 16)
        def _(r):
            cnt_v[r] = jnp.zeros((16,), jnp.float32)

        @pl.when(sid == 0)
        def _():
            pltpu.sync_copy(cnt_v, cnt_sh)
        plsc.subcore_barrier()

        ones = jnp.ones((16,), jnp.float32)
        for j in range(2):
            for t in range(8):
                v = idx_v[j, pl.ds(t * 16, 16)]
                row = lax.bitwise_and(lax.shift_right_logical(v, 4),
                                      (K // 16) - 1)
                lane = lax.bitwise_and(v, 15)
                plsc.addupdate_scatter(cnt_v, [row, lane], ones)
            for t in range(8):
                ridx_v[j, pl.ds(t * 16, 16)] = (
                    lax.iota(jnp.int32, 16) + (j * 128 + t * 16))

        for j in range(2):
            pltpu.sync_copy(cnt_v.at[pl.ds(j * 128, 128)],
                            cnt_sh.at[ridx_v.at[j]], add=True)

        for c in copies:
            c.wait()
        pltpu.sync_copy(rows_v, zq_hbm.at[pl.ds(base, ROWS_PER_W)])
        plsc.subcore_barrier()

        @pl.when(sid == 0)
        def _():
            pltpu.sync_copy(cnt_sh, cnt_hbm.at[cid])

    return k(cb_flat, idx2d)


def _final_body(res_ref, zq_ref, qin_ref, cnt_ref, ls_ref,
                qsum_ref, commit_ref, perp_ref):
    i = pl.program_id(0)
    rp = res_ref[...]
    q = rp + (zq_ref[...] - rp)
    qsum_ref[...] = qin_ref[...] + q

    @pl.when(i == NRB - 1)
    def _():
        cnt = cnt_ref[...]
        counts = cnt[:, 0, :] + cnt[:, 1, :]
        probs = counts * (1.0 / N)
        ent = jnp.sum(probs * jnp.log(probs + 1e-10), axis=1, keepdims=True)
        perps = jnp.exp(-ent)
        perp_ref[...] = jnp.sum(perps, keepdims=True) * (1.0 / L)
        ls = ls_ref[...] * (COMMIT / (N * D))
        commit_ref[...] = jnp.sum(ls, keepdims=True)

    @pl.when(i == 0)
    def _():
        commit_ref[...] = jnp.zeros((1, 1), jnp.float32)
        perp_ref[...] = jnp.zeros((1, 1), jnp.float32)


def _final(res, zq, qacc, counts, lsums):
    return pl.pallas_call(
        _final_body,
        grid=(NRB,),
        in_specs=[
            _ROWSPEC, _ROWSPEC, _ROWSPEC,
            pl.BlockSpec((L, SC_CORES, K), lambda i: (0, 0, 0)),
            pl.BlockSpec((1, L), lambda i: (0, 0)),
        ],
        out_specs=(_ROWSPEC, _SCALSPEC, _SCALSPEC),
        out_shape=(_ROW_SHAPE, _SCAL_SHAPE, _SCAL_SHAPE),
    )(res, zq, qacc, counts, lsums)


def kernel(z, codebooks):
    zf = z.reshape(N, D)
    cb_flat = codebooks.reshape(L * K, D)

    idx0, ls0 = _layer0(zf, codebooks)
    zq0, cnt0 = _sc_gather_hist(cb_flat, idx0.reshape(N // 128, 128), 0)

    res1, qacc1, idx1, ls1 = _layer1(zf, zq0, codebooks)
    zq1, cnt1 = _sc_gather_hist(cb_flat, idx1.reshape(N // 128, 128), 1)

    res2, qacc2, idx2, ls2 = _layerN(res1, zq1, qacc1, codebooks, 2)
    zq2, cnt2 = _sc_gather_hist(cb_flat, idx2.reshape(N // 128, 128), 2)

    res3, qacc3, idx3, ls3 = _layerN(res2, zq2, qacc2, codebooks, 3)
    zq3, cnt3 = _sc_gather_hist(cb_flat, idx3.reshape(N // 128, 128), 3)

    counts = jnp.stack([cnt0, cnt1, cnt2, cnt3]).reshape(L, SC_CORES, K)
    lsums = jnp.concatenate([ls0, ls1, ls2, ls3], axis=1)

    qsum, commit, perp = _final(res3, zq3, qacc3, counts, lsums)

    quantized_sum = qsum.reshape(B, S, D)
    indices = jnp.stack(
        [idx0.reshape(B, S), idx1.reshape(B, S),
         idx2.reshape(B, S), idx3.reshape(B, S)], axis=-1)
    total_commitment = commit[0, 0]
    codebook_loss = jnp.zeros((), jnp.float32)
    avg_perplexity = perp[0, 0]
    return (quantized_sum, indices, total_commitment, codebook_loss,
            avg_perplexity)
